# Initial kernel scaffold; baseline (speedup 1.0000x reference)
import jax
import jax.numpy as jnp
from jax.experimental import pallas as pl


def kernel(input_x, table):
    raise NotImplementedError("write your pallas kernel here")


# speedup vs baseline: 4.4435x; 4.4435x over previous
import functools
import jax
import jax.numpy as jnp
from jax import lax
from jax.experimental import pallas as pl
from jax.experimental.pallas import tpu as pltpu
from jax.experimental.pallas import tpu_sc as plsc

_CHUNK = 128


@functools.lru_cache(maxsize=None)
def _build(n_total, vocab, dim):
    info = plsc.get_sparse_core_info()
    nc, ns = info.num_cores, info.num_subcores
    nw = nc * ns
    assert n_total % (nw * _CHUNK) == 0
    nch = n_total // (nw * _CHUNK)

    mesh = plsc.VectorSubcoreMesh(core_axis_name="c", subcore_axis_name="s")

    @functools.partial(
        pl.kernel,
        mesh=mesh,
        out_type=jax.ShapeDtypeStruct((n_total, dim), jnp.float32),
        scratch_types=[
            pltpu.VMEM((nch, _CHUNK), jnp.int32),
            pltpu.VMEM((_CHUNK, dim), jnp.float32),
            pltpu.SemaphoreType.DMA,
        ],
    )
    def emb(idx_hbm, table_hbm, out_hbm, idx_v, rows_v, sem):
        wid = lax.axis_index("s") * nc + lax.axis_index("c")
        pltpu.sync_copy(idx_hbm.at[wid], idx_v)

        def step(j, carry):
            pltpu.async_copy(table_hbm.at[idx_v.at[j]], rows_v, sem).wait()
            pltpu.sync_copy(
                rows_v, out_hbm.at[pl.ds((wid * nch + j) * _CHUNK, _CHUNK)]
            )
            return carry

        lax.fori_loop(0, nch, step, 0)

    return emb, nw, nch


def kernel(input_x, table):
    b, h = input_x.shape
    v, d = table.shape
    n = b * h
    emb, nw, nch = _build(n, v, d)
    idx = input_x.reshape(nw, nch, _CHUNK).astype(jnp.int32)
    out = emb(idx, table).reshape(b, h, d)
    return (out, out)


# speedup vs baseline: 5.5424x; 1.2473x over previous
import functools
import jax
import jax.numpy as jnp
from jax import lax
from jax.experimental import pallas as pl
from jax.experimental.pallas import tpu as pltpu
from jax.experimental.pallas import tpu_sc as plsc

_CHUNK = 128
_NBUF = 4


@functools.lru_cache(maxsize=None)
def _build(n_total, vocab, dim):
    info = plsc.get_sparse_core_info()
    nc, ns = info.num_cores, info.num_subcores
    nw = nc * ns
    assert n_total % (nw * _CHUNK * _NBUF) == 0
    nch = n_total // (nw * _CHUNK)
    ngrp = nch // _NBUF

    mesh = plsc.VectorSubcoreMesh(core_axis_name="c", subcore_axis_name="s")

    @functools.partial(
        pl.kernel,
        mesh=mesh,
        out_type=jax.ShapeDtypeStruct((n_total, dim), jnp.float32),
        scratch_types=[
            pltpu.VMEM((nch, _CHUNK), jnp.int32),
            pltpu.VMEM((_NBUF, _CHUNK, dim), jnp.float32),
            pltpu.SemaphoreType.DMA((_NBUF,)),
            pltpu.SemaphoreType.DMA((_NBUF,)),
        ],
    )
    def emb(idx_hbm, table_hbm, out_hbm, idx_v, rows_v, gsem, wsem):
        wid = lax.axis_index("s") * nc + lax.axis_index("c")
        pltpu.sync_copy(idx_hbm.at[wid], idx_v)
        base = wid * nch

        def start_gather(j, b):
            pltpu.async_copy(table_hbm.at[idx_v.at[j]], rows_v.at[b], gsem.at[b])

        def wait_gather(j, b):
            pltpu.make_async_copy(
                table_hbm.at[idx_v.at[j]], rows_v.at[b], gsem.at[b]
            ).wait()

        def start_write(j, b):
            pltpu.async_copy(
                rows_v.at[b], out_hbm.at[pl.ds((base + j) * _CHUNK, _CHUNK)],
                wsem.at[b],
            )

        def wait_write(j, b):
            pltpu.make_async_copy(
                rows_v.at[b], out_hbm.at[pl.ds((base + j) * _CHUNK, _CHUNK)],
                wsem.at[b],
            ).wait()

        for b in range(_NBUF):
            start_gather(b, b)

        def group(g, carry):
            j0 = g * _NBUF
            for b in range(_NBUF):
                wait_gather(j0 + b, b)
                start_write(j0 + b, b)
            for b in range(_NBUF):
                wait_write(j0 + b, b)
                start_gather(j0 + _NBUF + b, b)
            return carry

        lax.fori_loop(0, ngrp - 1, group, 0)

        j0 = (ngrp - 1) * _NBUF
        for b in range(_NBUF):
            wait_gather(j0 + b, b)
            start_write(j0 + b, b)
        for b in range(_NBUF):
            wait_write(j0 + b, b)

    return emb, nw, nch


def kernel(input_x, table):
    b, h = input_x.shape
    v, d = table.shape
    n = b * h
    emb, nw, nch = _build(n, v, d)
    idx = input_x.reshape(nw, nch, _CHUNK).astype(jnp.int32)
    out = emb(idx, table).reshape(b, h, d)
    return (out, out)


# speedup vs baseline: 6.4210x; 1.1585x over previous
import functools
import jax
import jax.numpy as jnp
from jax import lax
from jax.experimental import pallas as pl
from jax.experimental.pallas import tpu as pltpu
from jax.experimental.pallas import tpu_sc as plsc

_CHUNK = 128
_NBUF = 4


@functools.lru_cache(maxsize=None)
def _build(n_total, vocab, dim):
    info = plsc.get_sparse_core_info()
    nc, ns = info.num_cores, info.num_subcores
    nw = nc * ns
    assert n_total % (nw * _CHUNK * _NBUF) == 0
    nch = n_total // (nw * _CHUNK)
    ngrp = nch // _NBUF

    mesh = plsc.VectorSubcoreMesh(core_axis_name="c", subcore_axis_name="s")

    out_struct = jax.ShapeDtypeStruct((n_total, dim), jnp.float32)

    @functools.partial(
        pl.kernel,
        mesh=mesh,
        out_type=(out_struct, out_struct),
        scratch_types=[
            pltpu.VMEM((nch, _CHUNK), jnp.int32),
            pltpu.VMEM((_NBUF, _CHUNK, dim), jnp.float32),
            pltpu.SemaphoreType.DMA((_NBUF,)),
            pltpu.SemaphoreType.DMA((_NBUF,)),
            pltpu.SemaphoreType.DMA((_NBUF,)),
        ],
    )
    def emb(idx_hbm, table_hbm, out_hbm, out2_hbm, idx_v, rows_v, gsem, wsem,
            w2sem):
        wid = lax.axis_index("s") * nc + lax.axis_index("c")
        pltpu.sync_copy(idx_hbm.at[wid], idx_v)
        base = wid * nch

        def start_gather(j, b):
            pltpu.async_copy(table_hbm.at[idx_v.at[j]], rows_v.at[b], gsem.at[b])

        def wait_gather(j, b):
            pltpu.make_async_copy(
                table_hbm.at[idx_v.at[j]], rows_v.at[b], gsem.at[b]
            ).wait()

        def start_write(j, b):
            pltpu.async_copy(
                rows_v.at[b], out_hbm.at[pl.ds((base + j) * _CHUNK, _CHUNK)],
                wsem.at[b],
            )
            pltpu.async_copy(
                rows_v.at[b], out2_hbm.at[pl.ds((base + j) * _CHUNK, _CHUNK)],
                w2sem.at[b],
            )

        def wait_write(j, b):
            pltpu.make_async_copy(
                rows_v.at[b], out_hbm.at[pl.ds((base + j) * _CHUNK, _CHUNK)],
                wsem.at[b],
            ).wait()
            pltpu.make_async_copy(
                rows_v.at[b], out2_hbm.at[pl.ds((base + j) * _CHUNK, _CHUNK)],
                w2sem.at[b],
            ).wait()

        for b in range(_NBUF):
            start_gather(b, b)

        def group(g, carry):
            j0 = g * _NBUF
            for b in range(_NBUF):
                wait_gather(j0 + b, b)
                start_write(j0 + b, b)
            for b in range(_NBUF):
                wait_write(j0 + b, b)
                start_gather(j0 + _NBUF + b, b)
            return carry

        lax.fori_loop(0, ngrp - 1, group, 0)

        j0 = (ngrp - 1) * _NBUF
        for b in range(_NBUF):
            wait_gather(j0 + b, b)
            start_write(j0 + b, b)
        for b in range(_NBUF):
            wait_write(j0 + b, b)

    return emb, nw, nch


def kernel(input_x, table):
    b, h = input_x.shape
    v, d = table.shape
    n = b * h
    emb, nw, nch = _build(n, v, d)
    idx = input_x.reshape(nw, nch, _CHUNK).astype(jnp.int32)
    out1, out2 = emb(idx, table)
    return (out1.reshape(b, h, d), out2.reshape(b, h, d))


# speedup vs baseline: 6.4435x; 1.0035x over previous
import functools
import jax
import jax.numpy as jnp
from jax import lax
from jax.experimental import pallas as pl
from jax.experimental.pallas import tpu as pltpu
from jax.experimental.pallas import tpu_sc as plsc

_CHUNK = 128
_NBUF = 5


@functools.lru_cache(maxsize=None)
def _build(n_total, vocab, dim):
    info = plsc.get_sparse_core_info()
    nc, ns = info.num_cores, info.num_subcores
    nw = nc * ns
    assert n_total % (nw * _CHUNK * _NBUF) == 0
    nch = n_total // (nw * _CHUNK)
    ngrp = nch // _NBUF

    mesh = plsc.VectorSubcoreMesh(core_axis_name="c", subcore_axis_name="s")

    out_struct = jax.ShapeDtypeStruct((n_total, dim), jnp.float32)

    @functools.partial(
        pl.kernel,
        mesh=mesh,
        out_type=(out_struct, out_struct),
        scratch_types=[
            pltpu.VMEM((nch, _CHUNK), jnp.int32),
            pltpu.VMEM((_NBUF, _CHUNK, dim), jnp.float32),
            pltpu.SemaphoreType.DMA((_NBUF,)),
            pltpu.SemaphoreType.DMA((_NBUF,)),
            pltpu.SemaphoreType.DMA((_NBUF,)),
        ],
    )
    def emb(idx_hbm, table_hbm, out_hbm, out2_hbm, idx_v, rows_v, gsem, wsem,
            w2sem):
        wid = lax.axis_index("s") * nc + lax.axis_index("c")
        pltpu.sync_copy(idx_hbm.at[wid], idx_v)
        base = wid * nch

        def start_gather(j, b):
            pltpu.async_copy(table_hbm.at[idx_v.at[j]], rows_v.at[b], gsem.at[b])

        def wait_gather(j, b):
            pltpu.make_async_copy(
                table_hbm.at[idx_v.at[j]], rows_v.at[b], gsem.at[b]
            ).wait()

        def start_write(j, b):
            pltpu.async_copy(
                rows_v.at[b], out_hbm.at[pl.ds((base + j) * _CHUNK, _CHUNK)],
                wsem.at[b],
            )
            pltpu.async_copy(
                rows_v.at[b], out2_hbm.at[pl.ds((base + j) * _CHUNK, _CHUNK)],
                w2sem.at[b],
            )

        def wait_write(j, b):
            pltpu.make_async_copy(
                rows_v.at[b], out_hbm.at[pl.ds((base + j) * _CHUNK, _CHUNK)],
                wsem.at[b],
            ).wait()
            pltpu.make_async_copy(
                rows_v.at[b], out2_hbm.at[pl.ds((base + j) * _CHUNK, _CHUNK)],
                w2sem.at[b],
            ).wait()

        for b in range(_NBUF):
            start_gather(b, b)

        def group(g, carry):
            j0 = g * _NBUF
            for b in range(_NBUF):
                wait_gather(j0 + b, b)
                start_write(j0 + b, b)
            for b in range(_NBUF):
                wait_write(j0 + b, b)
                start_gather(j0 + _NBUF + b, b)
            return carry

        lax.fori_loop(0, ngrp - 1, group, 0)

        j0 = (ngrp - 1) * _NBUF
        for b in range(_NBUF):
            wait_gather(j0 + b, b)
            start_write(j0 + b, b)
        for b in range(_NBUF):
            wait_write(j0 + b, b)

    return emb, nw, nch


def kernel(input_x, table):
    b, h = input_x.shape
    v, d = table.shape
    n = b * h
    emb, nw, nch = _build(n, v, d)
    idx = input_x.reshape(nw, nch, _CHUNK).astype(jnp.int32)
    out1, out2 = emb(idx, table)
    return (out1.reshape(b, h, d), out2.reshape(b, h, d))
